# Initial kernel scaffold; baseline (speedup 1.0000x reference)
import jax
import jax.numpy as jnp
from jax.experimental import pallas as pl


def kernel(graph_embeddings, retrieval_embeddings, pred_W, pred_b, adapter_W, retrieval_y):
    raise NotImplementedError("write your pallas kernel here")


# speedup vs baseline: 4.4125x; 4.4125x over previous
import functools

import jax
import jax.numpy as jnp
from jax.experimental import pallas as pl
from jax.experimental.pallas import tpu as pltpu

B, D, C, K = 1024, 256, 128, 10
BB = 256
NEG = -1e30


def _tc_body(x_ref, retr_ref, pw_ref, pb_ref, aw_ref, yt_ref, out_ref):
    x = x_ref[...]
    logits = jnp.dot(x, pw_ref[...], preferred_element_type=jnp.float32)
    logits = logits + pb_ref[...][None, :]
    m = jnp.max(logits, axis=1, keepdims=True)
    e = jnp.exp(logits - m)
    g = e / jnp.sum(e, axis=1, keepdims=True)

    q = jnp.dot(x, aw_ref[...], preferred_element_type=jnp.float32)
    s0 = jnp.sum(q * x, axis=1)
    sk = jnp.sum(retr_ref[...] * q[None, :, :], axis=2)
    scores = jnp.concatenate([s0[None, :], sk], axis=0)

    sm = jnp.max(scores, axis=0, keepdims=True)
    se = jnp.exp(scores - sm)
    att = se / jnp.sum(se, axis=0, keepdims=True)

    out = att[0][:, None] * g
    cls = jax.lax.broadcasted_iota(jnp.int32, (1, C), 1)
    yt = yt_ref[...]
    for k in range(K):
        hit = (yt[:, k][:, None] == cls).astype(jnp.float32)
        out = out + att[k + 1][:, None] * hit
    out_ref[...] = out * jnp.float32(C)


@functools.partial(jax.jit, static_argnames=("interpret",))
def _run(graph_embeddings, retrieval_embeddings, pred_W, pred_b, adapter_W,
         yt, interpret=False):
    grid = (B // BB,)
    return pl.pallas_call(
        _tc_body,
        grid=grid,
        in_specs=[
            pl.BlockSpec((BB, D), lambda i: (i, 0)),
            pl.BlockSpec((K, BB, D), lambda i: (0, i, 0)),
            pl.BlockSpec((D, C), lambda i: (0, 0)),
            pl.BlockSpec((C,), lambda i: (0,)),
            pl.BlockSpec((D, D), lambda i: (0, 0)),
            pl.BlockSpec((BB, 16), lambda i: (i, 0)),
        ],
        out_specs=pl.BlockSpec((BB, C), lambda i: (i, 0)),
        out_shape=jax.ShapeDtypeStruct((B, C), jnp.float32),
        interpret=interpret,
    )(graph_embeddings, retrieval_embeddings, pred_W, pred_b, adapter_W, yt)


def kernel(graph_embeddings, retrieval_embeddings, pred_W, pred_b, adapter_W,
           retrieval_y):
    yt = jnp.transpose(retrieval_y.astype(jnp.int32))
    yt = jnp.pad(yt, ((0, 0), (0, 16 - K)), constant_values=-1)
    return _run(graph_embeddings, retrieval_embeddings, pred_W, pred_b,
                adapter_W, yt)
